# baseline (speedup 1.0000x reference)
import functools

import jax
import jax.numpy as jnp
from jax import lax
from jax.experimental import pallas as pl
from jax.experimental.pallas import tpu as pltpu
from jax.experimental.pallas import tpu_sc as plsc

_TOP_K = 256
_STEP = 32
_LAST = 119
_NC = 2
_NS = 16
_L = 16
_G = 8
_COVER = 512
_CHUNKS = _TOP_K // _L


def _zero_rows(st_v, ncols):
    zeros16 = jnp.zeros((_L,), jnp.float32)

    def zbody(i, carry):
        for j in range(_G):
            st_v[j, pl.ds(i * _L, _L)] = zeros16
        return carry

    lax.fori_loop(0, ncols // _L, zbody, 0)


def _sc_body(x_hbm, out_hbm, xg_v, st_v):
    B, Q, K = x_hbm.shape
    nw = _NC * _NS
    rows_per_w = (B * Q) // nw
    n_groups = rows_per_w // _G
    wid = lax.axis_index("s") * _NC + lax.axis_index("c")
    b = wid // (Q // rows_per_w)
    q_base = (wid % (Q // rows_per_w)) * rows_per_w

    _zero_rows(st_v, K)

    def group_body(g, carry):
        q0 = q_base + g * _G
        q0 = pl.multiple_of(q0, _G)
        c0 = jnp.minimum(_STEP * q0, K - _COVER)
        c0 = pl.multiple_of(c0, 128)

        pltpu.sync_copy(x_hbm.at[b, pl.ds(q0, _G), pl.ds(c0, _COVER)], xg_v)

        for j in range(_G):
            q = q0 + j
            loc = _STEP * jnp.minimum(q, _LAST) - c0
            xs = [xg_v[j, pl.ds(loc + i * _L, _L)] for i in range(_CHUNKS)]
            m = jnp.max(functools.reduce(jnp.maximum, xs))
            es = [jnp.exp(x - m) for x in xs]
            total = jnp.sum(functools.reduce(lambda u, v: u + v, es))
            dst = _STEP * jnp.minimum(q, _LAST)
            for i in range(_CHUNKS):
                st_v[j, pl.ds(dst + i * _L, _L)] = es[i] / total

        pltpu.sync_copy(st_v, out_hbm.at[b, pl.ds(q0, _G)])

        zeros16 = jnp.zeros((_L,), jnp.float32)
        for j in range(_G):
            q = q0 + j
            dst = _STEP * jnp.minimum(q, _LAST)
            for i in range(_CHUNKS):
                st_v[j, pl.ds(dst + i * _L, _L)] = zeros16
        return carry

    lax.fori_loop(0, n_groups, group_body, 0)


def kernel(X):
    B, Q, K = X.shape
    mesh = plsc.VectorSubcoreMesh(core_axis_name="c", subcore_axis_name="s")
    f = pl.kernel(
        _sc_body,
        out_type=jax.ShapeDtypeStruct((B, Q, K), jnp.float32),
        mesh=mesh,
        scratch_types=[
            pltpu.VMEM((_G, _COVER), jnp.float32),
            pltpu.VMEM((_G, K), jnp.float32),
        ],
        compiler_params=pltpu.CompilerParams(needs_layout_passes=False),
    )
    return f(X)


# speedup vs baseline: 1.3298x; 1.3298x over previous
import functools

import jax
import jax.numpy as jnp
from jax import lax
from jax.experimental import pallas as pl
from jax.experimental.pallas import tpu as pltpu
from jax.experimental.pallas import tpu_sc as plsc

_TOP_K = 256
_STEP = 32
_LAST = 119
_NC = 2
_NS = 16
_L = 16
_G = 8
_COVER = 512
_CHUNKS = _TOP_K // _L


def _sc_body(x_hbm, out_hbm, xg0, xg1, st0, st1, isem0, isem1, osem0, osem1):
    B, Q, K = x_hbm.shape
    nw = _NC * _NS
    rows_per_w = (B * Q) // nw
    n_groups = rows_per_w // _G
    wid = lax.axis_index("s") * _NC + lax.axis_index("c")
    b = wid // (Q // rows_per_w)
    q_base = (wid % (Q // rows_per_w)) * rows_per_w
    xgs, sts = (xg0, xg1), (st0, st1)
    isems, osems = (isem0, isem1), (osem0, osem1)

    zeros16 = jnp.zeros((_L,), jnp.float32)

    def zero_full(st_v):
        def zbody(i, carry):
            for j in range(_G):
                st_v[j, pl.ds(i * _L, _L)] = zeros16
            return carry

        lax.fori_loop(0, K // _L, zbody, 0)

    zero_full(st0)
    zero_full(st1)

    def group_q0(g):
        return pl.multiple_of(q_base + g * _G, _G)

    def group_c0(q0):
        return pl.multiple_of(jnp.minimum(_STEP * q0, K - _COVER), 128)

    def gather_desc(g, k):
        q0 = group_q0(g)
        c0 = group_c0(q0)
        return pltpu.make_async_copy(
            x_hbm.at[b, pl.ds(q0, _G), pl.ds(c0, _COVER)], xgs[k], isems[k])

    def out_desc(g, k):
        q0 = group_q0(g)
        return pltpu.make_async_copy(
            sts[k], out_hbm.at[b, pl.ds(q0, _G)], osems[k])

    gather_desc(0, 0).start()

    def pair_body(gp, carry):
        for k in (0, 1):
            g = 2 * gp + k
            xg_v, st_v = xgs[k], sts[k]

            @pl.when(g + 1 < n_groups)
            def _():
                gather_desc(g + 1, k ^ 1).start()

            q0 = group_q0(g)
            c0 = group_c0(q0)
            gather_desc(g, k).wait()

            @pl.when(g >= 2)
            def _():
                out_desc(g, k).wait()
                q0p = q0 - 2 * _G
                for j in range(_G):
                    dstp = _STEP * jnp.minimum(q0p + j, _LAST)
                    for i in range(_CHUNKS):
                        st_v[j, pl.ds(dstp + i * _L, _L)] = zeros16

            for j in range(_G):
                q = q0 + j
                loc = _STEP * jnp.minimum(q, _LAST) - c0
                xs = [xg_v[j, pl.ds(loc + i * _L, _L)] for i in range(_CHUNKS)]
                m = jnp.max(functools.reduce(jnp.maximum, xs))
                es = [jnp.exp(x - m) for x in xs]
                total = jnp.sum(functools.reduce(lambda u, v: u + v, es))
                dst = _STEP * jnp.minimum(q, _LAST)
                for i in range(_CHUNKS):
                    st_v[j, pl.ds(dst + i * _L, _L)] = es[i] / total

            out_desc(g, k).start()
        return carry

    lax.fori_loop(0, n_groups // 2, pair_body, 0)

    for k in (0, 1):
        out_desc(n_groups - 2 + k, k).wait()


def kernel(X):
    B, Q, K = X.shape
    mesh = plsc.VectorSubcoreMesh(core_axis_name="c", subcore_axis_name="s")
    f = pl.kernel(
        _sc_body,
        out_type=jax.ShapeDtypeStruct((B, Q, K), jnp.float32),
        mesh=mesh,
        scratch_types=[
            pltpu.VMEM((_G, _COVER), jnp.float32),
            pltpu.VMEM((_G, _COVER), jnp.float32),
            pltpu.VMEM((_G, K), jnp.float32),
            pltpu.VMEM((_G, K), jnp.float32),
            pltpu.SemaphoreType.DMA,
            pltpu.SemaphoreType.DMA,
            pltpu.SemaphoreType.DMA,
            pltpu.SemaphoreType.DMA,
        ],
        compiler_params=pltpu.CompilerParams(needs_layout_passes=False),
    )
    return f(X)


# speedup vs baseline: 1.3935x; 1.0479x over previous
import functools

import jax
import jax.numpy as jnp
from jax import lax
from jax.experimental import pallas as pl
from jax.experimental.pallas import tpu as pltpu
from jax.experimental.pallas import tpu_sc as plsc

_TOP_K = 256
_STEP = 32
_LAST = 119
_FIX0 = _STEP * _LAST
_FCOV0 = 3712
_FCOVW = 384
_SCOVW = 512
_NC = 2
_NS = 16
_L = 16
_G = 8
_CHUNKS = _TOP_K // _L
_Q0_SLIDE_MAX = 112


def _softmax_row(xs):
    m = jnp.max(functools.reduce(jnp.maximum, xs))
    es = [jnp.exp(x - m) for x in xs]
    total = jnp.sum(functools.reduce(lambda u, v: u + v, es))
    return [e / total for e in es]


def _sc_body(x_hbm, out_hbm, xg0, xg1, st0, st1, isem0, isem1, osem0, osem1):
    B, Q, K = x_hbm.shape
    nw = _NC * _NS
    rows_per_w = (B * Q) // nw
    n_groups = rows_per_w // _G
    wid = lax.axis_index("s") * _NC + lax.axis_index("c")
    b = wid // (Q // rows_per_w)
    q_base = (wid % (Q // rows_per_w)) * rows_per_w
    xgs, sts = (xg0, xg1), (st0, st1)
    isems, osems = (isem0, isem1), (osem0, osem1)

    zeros16 = jnp.zeros((_L,), jnp.float32)

    def zero_full(st_v):
        def zbody(i, carry):
            for j in range(_G):
                st_v[j, pl.ds(i * _L, _L)] = zeros16
            return carry

        lax.fori_loop(0, K // _L, zbody, 0)

    zero_full(st0)
    zero_full(st1)

    def group_q0(g):
        return pl.multiple_of(q_base + g * _G, _G)

    def slide_gather_desc(q0, k):
        c0 = pl.multiple_of(_STEP * q0, 128)
        return pltpu.make_async_copy(
            x_hbm.at[b, pl.ds(q0, _G), pl.ds(c0, _SCOVW)], xgs[k], isems[k])

    def fixed_gather_desc(q0, k):
        return pltpu.make_async_copy(
            x_hbm.at[b, pl.ds(q0, _G), pl.ds(_FCOV0, _FCOVW)],
            xgs[k].at[:, pl.ds(0, _FCOVW)], isems[k])

    def start_gather(g, k):
        q0 = group_q0(g)

        @pl.when(q0 <= _Q0_SLIDE_MAX)
        def _():
            slide_gather_desc(q0, k).start()

        @pl.when(q0 > _Q0_SLIDE_MAX)
        def _():
            fixed_gather_desc(q0, k).start()

    def wait_gather(g, k):
        q0 = group_q0(g)

        @pl.when(q0 <= _Q0_SLIDE_MAX)
        def _():
            slide_gather_desc(q0, k).wait()

        @pl.when(q0 > _Q0_SLIDE_MAX)
        def _():
            fixed_gather_desc(q0, k).wait()

    def out_desc(g, k):
        q0 = group_q0(g)
        return pltpu.make_async_copy(
            sts[k], out_hbm.at[b, pl.ds(q0, _G)], osems[k])

    start_gather(0, 0)

    def pair_body(gp, carry):
        for k in (0, 1):
            g = 2 * gp + k
            xg_v, st_v = xgs[k], sts[k]

            @pl.when(g + 1 < n_groups)
            def _():
                start_gather(g + 1, k ^ 1)

            q0 = group_q0(g)
            wait_gather(g, k)

            @pl.when(g >= 2)
            def _():
                out_desc(g, k).wait()

            @pl.when((g >= 2) & (q0 - 2 * _G <= _Q0_SLIDE_MAX))
            def _():
                q0p = q0 - 2 * _G
                for j in range(_G):
                    dstp = _STEP * jnp.minimum(q0p + j, _LAST)
                    for i in range(_CHUNKS):
                        st_v[j, pl.ds(dstp + i * _L, _L)] = zeros16

            @pl.when(q0 <= _Q0_SLIDE_MAX)
            def _():
                for j in range(_G):
                    xs = [xg_v[j, pl.ds(_STEP * j + i * _L, _L)]
                          for i in range(_CHUNKS)]
                    ys = _softmax_row(xs)
                    dst = _STEP * q0 + _STEP * j
                    for i in range(_CHUNKS):
                        st_v[j, pl.ds(dst + i * _L, _L)] = ys[i]

            @pl.when(q0 > _Q0_SLIDE_MAX)
            def _():
                for j in range(_G):
                    xs = [xg_v[j, pl.ds(_FIX0 - _FCOV0 + i * _L, _L)]
                          for i in range(_CHUNKS)]
                    ys = _softmax_row(xs)
                    for i in range(_CHUNKS):
                        st_v[j, pl.ds(_FIX0 + i * _L, _L)] = ys[i]

            out_desc(g, k).start()
        return carry

    lax.fori_loop(0, n_groups // 2, pair_body, 0)

    for k in (0, 1):
        out_desc(n_groups - 2 + k, k).wait()


def kernel(X):
    B, Q, K = X.shape
    mesh = plsc.VectorSubcoreMesh(core_axis_name="c", subcore_axis_name="s")
    f = pl.kernel(
        _sc_body,
        out_type=jax.ShapeDtypeStruct((B, Q, K), jnp.float32),
        mesh=mesh,
        scratch_types=[
            pltpu.VMEM((_G, _SCOVW), jnp.float32),
            pltpu.VMEM((_G, _SCOVW), jnp.float32),
            pltpu.VMEM((_G, K), jnp.float32),
            pltpu.VMEM((_G, K), jnp.float32),
            pltpu.SemaphoreType.DMA,
            pltpu.SemaphoreType.DMA,
            pltpu.SemaphoreType.DMA,
            pltpu.SemaphoreType.DMA,
        ],
        compiler_params=pltpu.CompilerParams(needs_layout_passes=False),
    )
    return f(X)


# speedup vs baseline: 1.4134x; 1.0143x over previous
import functools

import jax
import jax.numpy as jnp
from jax import lax
from jax.experimental import pallas as pl
from jax.experimental.pallas import tpu as pltpu
from jax.experimental.pallas import tpu_sc as plsc

_TOP_K = 256
_STEP = 32
_LAST = 119
_FIX0 = _STEP * _LAST
_FCOV0 = 3712
_FCOVW = 384
_SCOVW = 512
_NC = 2
_NS = 16
_L = 16
_G = 8
_CHUNKS = _TOP_K // _L
_Q0_SLIDE_MAX = 112


def _softmax_row(xs):
    m = jnp.max(functools.reduce(jnp.maximum, xs))
    es = [jnp.exp(x - m) for x in xs]
    total = jnp.sum(functools.reduce(lambda u, v: u + v, es))
    return [e / total for e in es]


def _sc_body(x_hbm, out_hbm, xg0, xg1, st0, st1, isem0, isem1, osem0, osem1):
    B, Q, K = x_hbm.shape
    nw = _NC * _NS
    rows_per_w = (B * Q) // nw
    n_groups = rows_per_w // _G
    wid = lax.axis_index("s") * _NC + lax.axis_index("c")
    b = wid // (Q // rows_per_w)
    q_base = (wid % (Q // rows_per_w)) * rows_per_w
    xgs, sts = (xg0, xg1), (st0, st1)
    isems, osems = (isem0, isem1), (osem0, osem1)

    zeros16 = jnp.zeros((_L,), jnp.float32)

    def zero_full(st_v):
        def zbody(i, carry):
            for j in range(_G):
                st_v[j, pl.ds(i * _L, _L)] = zeros16
            return carry

        lax.fori_loop(0, K // _L, zbody, 0)

    def group_q0(g):
        return pl.multiple_of(q_base + g * _G, _G)

    def slide_gather_desc(q0, k):
        c0 = pl.multiple_of(_STEP * q0, 128)
        return pltpu.make_async_copy(
            x_hbm.at[b, pl.ds(q0, _G), pl.ds(c0, _SCOVW)], xgs[k], isems[k])

    def fixed_gather_desc(q0, k):
        return pltpu.make_async_copy(
            x_hbm.at[b, pl.ds(q0, _G), pl.ds(_FCOV0, _FCOVW)],
            xgs[k].at[:, pl.ds(0, _FCOVW)], isems[k])

    def start_gather(g, k):
        q0 = group_q0(g)

        @pl.when(q0 <= _Q0_SLIDE_MAX)
        def _():
            slide_gather_desc(q0, k).start()

        @pl.when(q0 > _Q0_SLIDE_MAX)
        def _():
            fixed_gather_desc(q0, k).start()

    def wait_gather(g, k):
        q0 = group_q0(g)

        @pl.when(q0 <= _Q0_SLIDE_MAX)
        def _():
            slide_gather_desc(q0, k).wait()

        @pl.when(q0 > _Q0_SLIDE_MAX)
        def _():
            fixed_gather_desc(q0, k).wait()

    def out_desc(g, k):
        q0 = group_q0(g)
        return pltpu.make_async_copy(
            sts[k], out_hbm.at[b, pl.ds(q0, _G)], osems[k])

    start_gather(0, 0)
    zero_full(st0)
    zero_full(st1)

    def pair_body(gp, carry):
        for k in (0, 1):
            g = 2 * gp + k
            xg_v, st_v = xgs[k], sts[k]

            @pl.when(g + 1 < n_groups)
            def _():
                start_gather(g + 1, k ^ 1)

            q0 = group_q0(g)
            wait_gather(g, k)

            @pl.when(g >= 2)
            def _():
                out_desc(g, k).wait()

            @pl.when((g >= 2) & (q0 - 2 * _G <= _Q0_SLIDE_MAX))
            def _():
                q0p = q0 - 2 * _G
                for j in range(_G):
                    dstp = _STEP * jnp.minimum(q0p + j, _LAST)
                    for i in range(_CHUNKS):
                        st_v[j, pl.ds(dstp + i * _L, _L)] = zeros16

            @pl.when(q0 <= _Q0_SLIDE_MAX)
            def _():
                for j in range(_G):
                    xs = [xg_v[j, pl.ds(_STEP * j + i * _L, _L)]
                          for i in range(_CHUNKS)]
                    ys = _softmax_row(xs)
                    dst = _STEP * q0 + _STEP * j
                    for i in range(_CHUNKS):
                        st_v[j, pl.ds(dst + i * _L, _L)] = ys[i]

            @pl.when(q0 > _Q0_SLIDE_MAX)
            def _():
                for j in range(_G):
                    xs = [xg_v[j, pl.ds(_FIX0 - _FCOV0 + i * _L, _L)]
                          for i in range(_CHUNKS)]
                    ys = _softmax_row(xs)
                    for i in range(_CHUNKS):
                        st_v[j, pl.ds(_FIX0 + i * _L, _L)] = ys[i]

            out_desc(g, k).start()
        return carry

    lax.fori_loop(0, n_groups // 2, pair_body, 0)

    for k in (0, 1):
        out_desc(n_groups - 2 + k, k).wait()


def kernel(X):
    B, Q, K = X.shape
    mesh = plsc.VectorSubcoreMesh(core_axis_name="c", subcore_axis_name="s")
    f = pl.kernel(
        _sc_body,
        out_type=jax.ShapeDtypeStruct((B, Q, K), jnp.float32),
        mesh=mesh,
        scratch_types=[
            pltpu.VMEM((_G, _SCOVW), jnp.float32),
            pltpu.VMEM((_G, _SCOVW), jnp.float32),
            pltpu.VMEM((_G, K), jnp.float32),
            pltpu.VMEM((_G, K), jnp.float32),
            pltpu.SemaphoreType.DMA,
            pltpu.SemaphoreType.DMA,
            pltpu.SemaphoreType.DMA,
            pltpu.SemaphoreType.DMA,
        ],
        compiler_params=pltpu.CompilerParams(needs_layout_passes=False),
    )
    return f(X)
